# Initial kernel scaffold; baseline (speedup 1.0000x reference)
import jax
import jax.numpy as jnp
from jax.experimental import pallas as pl


def kernel(x, W_d, b_d, W_a):
    raise NotImplementedError("write your pallas kernel here")


# speedup vs baseline: 2.9987x; 2.9987x over previous
import functools

import jax
import jax.numpy as jnp
from jax import lax
from jax.experimental import pallas as pl
from jax.experimental.pallas import tpu as pltpu
from jax.experimental.pallas import tpu_sc as plsc

L = 16
NW = 32
CHUNK = 1568


def _union_body(n_points, n_chunks, x_hbm, wd_hbm, b_hbm, wa_hbm,
                minv_hbm, sel_hbm, wd_v, b_v, wa_v, x_v, minv_v, sel_v):
    info = plsc.get_sparse_core_info()
    nc = info.num_cores
    wid = lax.axis_index("s") * nc + lax.axis_index("c")
    span = n_chunks * CHUNK

    pltpu.sync_copy(wd_hbm, wd_v)
    pltpu.sync_copy(b_hbm, b_v)
    pltpu.sync_copy(wa_hbm, wa_v)

    wcol = [wd_v[pl.ds(16 * d, 16)] for d in range(3)]
    bvec = b_v[...]
    wd = [[wcol[d][k] for d in range(3)] for k in range(16)]
    bs = [bvec[k] for k in range(16)]

    groups = CHUNK // L

    def chunk_body(c, carry):
        start = jnp.minimum(wid * span + c * CHUNK, n_points - CHUNK)
        start = pl.multiple_of(start, 8)
        for d in range(3):
            pltpu.sync_copy(x_hbm.at[pl.ds(d * n_points + start, CHUNK)],
                            x_v.at[pl.ds(d * CHUNK, CHUNK)])

        def group_body(g, gcarry):
            gb = pl.multiple_of(g * L, 8)
            x0 = x_v[pl.ds(gb, L)]
            x1 = x_v[pl.ds(CHUNK + gb, L)]
            x2 = x_v[pl.ds(2 * CHUNK + gb, L)]

            minv = x0 * wd[0][0] + x1 * wd[0][1] + x2 * wd[0][2] + bs[0]
            idxv = jnp.zeros((L,), jnp.int32)
            for k in range(1, 16):
                t = x0 * wd[k][0] + x1 * wd[k][1] + x2 * wd[k][2] + bs[k]
                m = t < minv
                idxv = jnp.where(m, k, idxv)
                minv = jnp.where(m, t, minv)
            minv_v[pl.ds(gb, L)] = minv

            base = idxv * 48
            for p in range(L):
                jb = pl.multiple_of(base[p], 16)
                w0 = wa_v[pl.ds(jb, L)]
                w1 = wa_v[pl.ds(jb + 16, L)]
                w2 = wa_v[pl.ds(jb + 32, L)]
                sv = x0[p] * w0 + x1[p] * w1 + x2[p] * w2
                ob = pl.multiple_of((gb + p) * 16, 16)
                sel_v[pl.ds(ob, L)] = sv
            return gcarry

        lax.fori_loop(0, groups, group_body, 0, unroll=False)
        pltpu.sync_copy(minv_v, minv_hbm.at[pl.ds(start, CHUNK)])
        pltpu.sync_copy(sel_v, sel_hbm.at[pl.ds(start * 16, CHUNK * 16)])
        return carry

    lax.fori_loop(0, n_chunks, chunk_body, 0, unroll=False)


def kernel(x, W_d, b_d, W_a):
    n, d = x.shape
    k = W_d.shape[0]
    a = W_a.shape[2]
    assert (d, k, a) == (3, 16, 16)
    n_chunks = -(-n // (NW * CHUNK))

    body = functools.partial(_union_body, n, n_chunks)
    run = pl.kernel(
        body,
        out_type=(
            jax.ShapeDtypeStruct((n,), jnp.float32),
            jax.ShapeDtypeStruct((n * 16,), jnp.float32),
        ),
        mesh=plsc.VectorSubcoreMesh(core_axis_name="c", subcore_axis_name="s"),
        scratch_types=[
            pltpu.VMEM((48,), jnp.float32),
            pltpu.VMEM((16,), jnp.float32),
            pltpu.VMEM((768,), jnp.float32),
            pltpu.VMEM((CHUNK * 3,), jnp.float32),
            pltpu.VMEM((CHUNK,), jnp.float32),
            pltpu.VMEM((CHUNK * 16,), jnp.float32),
        ],
    )
    def _bf16_round(v):
        u = lax.bitcast_convert_type(v, jnp.uint32)
        r = (u + jnp.uint32(0x7FFF) + ((u >> 16) & jnp.uint32(1))) \
            & jnp.uint32(0xFFFF0000)
        return lax.bitcast_convert_type(r, jnp.float32)

    xb = _bf16_round(x)
    wdb = _bf16_round(W_d)
    wab = _bf16_round(W_a)
    min_vals, sel_flat = run(
        xb.T.reshape(-1),
        wdb.T.reshape(-1),
        b_d,
        wab.reshape(-1),
    )
    return min_vals, sel_flat.reshape(n, 16)


# speedup vs baseline: 3.2168x; 1.0727x over previous
import functools

import jax
import jax.numpy as jnp
from jax import lax
from jax.experimental import pallas as pl
from jax.experimental.pallas import tpu as pltpu
from jax.experimental.pallas import tpu_sc as plsc

L = 16
NW = 32
CHUNK = 2608


def _union_body(n_points, n_chunks, x_hbm, wd_hbm, b_hbm, wa_hbm,
                minv_hbm, sel_hbm,
                wd_v, b_v, wa_v,
                xa0, xa1, xa2, mva, sela,
                xb0, xb1, xb2, mvb, selb,
                sin_a, sin_b, sout_a, sout_b):
    info = plsc.get_sparse_core_info()
    nc = info.num_cores
    wid = lax.axis_index("s") * nc + lax.axis_index("c")
    span = n_chunks * CHUNK

    pltpu.sync_copy(wd_hbm, wd_v)
    pltpu.sync_copy(b_hbm, b_v)
    pltpu.sync_copy(wa_hbm, wa_v)

    wcol = [wd_v[pl.ds(16 * d, 16)] for d in range(3)]
    bvec = b_v[...]
    wd = [[wcol[d][k] for d in range(3)] for k in range(16)]
    bs = [bvec[k] for k in range(16)]

    groups = CHUNK // L

    def cstart(c):
        s = jnp.minimum(wid * span + c * CHUNK, n_points - CHUNK)
        return pl.multiple_of(s, 8)

    def fire_in(c, bufs, sem):
        s = cstart(c)
        for d, dst in enumerate(bufs):
            pltpu.async_copy(x_hbm.at[pl.ds(d * n_points + s, CHUNK)],
                             dst, sem)

    def wait_in(bufs, sem):
        for dst in bufs:
            pltpu.make_async_copy(x_hbm.at[pl.ds(0, CHUNK)], dst, sem).wait()

    def fire_out(c, mv, sel, sem):
        s = cstart(c)
        pltpu.async_copy(mv, minv_hbm.at[pl.ds(s, CHUNK)], sem)
        pltpu.async_copy(sel, sel_hbm.at[pl.ds(s * 16, CHUNK * 16)], sem)

    def wait_out(mv, sel, sem):
        pltpu.make_async_copy(mv, minv_hbm.at[pl.ds(0, CHUNK)], sem).wait()
        pltpu.make_async_copy(
            sel, sel_hbm.at[pl.ds(0, CHUNK * 16)], sem).wait()

    def compute(x0b, x1b, x2b, mv, sel):
        def group_body(g, gcarry):
            gb = pl.multiple_of(g * L, 8)
            x0 = x0b[pl.ds(gb, L)]
            x1 = x1b[pl.ds(gb, L)]
            x2 = x2b[pl.ds(gb, L)]

            minv = x0 * wd[0][0] + x1 * wd[0][1] + x2 * wd[0][2] + bs[0]
            idxv = jnp.zeros((L,), jnp.int32)
            for k in range(1, 16):
                t = x0 * wd[k][0] + x1 * wd[k][1] + x2 * wd[k][2] + bs[k]
                m = t < minv
                idxv = jnp.where(m, k, idxv)
                minv = jnp.where(m, t, minv)
            mv[pl.ds(gb, L)] = minv

            base = idxv * 48
            for p in range(L):
                jb = pl.multiple_of(base[p], 16)
                w0 = wa_v[pl.ds(jb, L)]
                w1 = wa_v[pl.ds(jb + 16, L)]
                w2 = wa_v[pl.ds(jb + 32, L)]
                sv = x0[p] * w0 + x1[p] * w1 + x2[p] * w2
                ob = pl.multiple_of((gb + p) * 16, 16)
                sel[pl.ds(ob, L)] = sv
            return gcarry

        lax.fori_loop(0, groups, group_body, 0, unroll=False)

    bufa = (xa0, xa1, xa2)
    bufb = (xb0, xb1, xb2)
    fire_in(0, bufa, sin_a)

    def body2(c2, carry):
        c = 2 * c2
        wait_in(bufa, sin_a)

        @pl.when(c + 1 < n_chunks)
        def _():
            fire_in(c + 1, bufb, sin_b)

        @pl.when(c2 >= 1)
        def _():
            wait_out(mva, sela, sout_a)

        compute(xa0, xa1, xa2, mva, sela)
        fire_out(c, mva, sela, sout_a)

        wait_in(bufb, sin_b)

        @pl.when(c + 2 < n_chunks)
        def _():
            fire_in(c + 2, bufa, sin_a)

        @pl.when(c2 >= 1)
        def _():
            wait_out(mvb, selb, sout_b)

        compute(xb0, xb1, xb2, mvb, selb)
        fire_out(c + 1, mvb, selb, sout_b)
        return carry

    lax.fori_loop(0, n_chunks // 2, body2, 0, unroll=False)
    wait_out(mva, sela, sout_a)
    wait_out(mvb, selb, sout_b)


def kernel(x, W_d, b_d, W_a):
    n, d = x.shape
    k = W_d.shape[0]
    a = W_a.shape[2]
    assert (d, k, a) == (3, 16, 16)
    n_chunks = -(-n // (NW * CHUNK))
    n_chunks += n_chunks % 2

    body = functools.partial(_union_body, n, n_chunks)
    xbuf = lambda: pltpu.VMEM((CHUNK,), jnp.float32)
    run = pl.kernel(
        body,
        out_type=(
            jax.ShapeDtypeStruct((n,), jnp.float32),
            jax.ShapeDtypeStruct((n * 16,), jnp.float32),
        ),
        mesh=plsc.VectorSubcoreMesh(core_axis_name="c", subcore_axis_name="s"),
        scratch_types=[
            pltpu.VMEM((48,), jnp.float32),
            pltpu.VMEM((16,), jnp.float32),
            pltpu.VMEM((768,), jnp.float32),
            xbuf(), xbuf(), xbuf(),
            pltpu.VMEM((CHUNK,), jnp.float32),
            pltpu.VMEM((CHUNK * 16,), jnp.float32),
            xbuf(), xbuf(), xbuf(),
            pltpu.VMEM((CHUNK,), jnp.float32),
            pltpu.VMEM((CHUNK * 16,), jnp.float32),
            pltpu.SemaphoreType.DMA,
            pltpu.SemaphoreType.DMA,
            pltpu.SemaphoreType.DMA,
            pltpu.SemaphoreType.DMA,
        ],
    )
    def _bf16_round(v):
        u = lax.bitcast_convert_type(v, jnp.uint32)
        r = (u + jnp.uint32(0x7FFF) + ((u >> 16) & jnp.uint32(1))) \
            & jnp.uint32(0xFFFF0000)
        return lax.bitcast_convert_type(r, jnp.float32)

    xb = _bf16_round(x)
    wdb = _bf16_round(W_d)
    wab = _bf16_round(W_a)
    min_vals, sel_flat = run(
        xb.T.reshape(-1),
        wdb.T.reshape(-1),
        b_d,
        wab.reshape(-1),
    )
    return min_vals, sel_flat.reshape(n, 16)
